# Initial kernel scaffold; baseline (speedup 1.0000x reference)
import jax
import jax.numpy as jnp
from jax.experimental import pallas as pl


def kernel(input, users_matrix, user_id):
    raise NotImplementedError("write your pallas kernel here")


# speedup vs baseline: 3.2476x; 3.2476x over previous
import functools

import jax
import jax.numpy as jnp
from jax import lax
from jax.experimental import pallas as pl
from jax.experimental.pallas import tpu as pltpu
from jax.experimental.pallas import tpu_sc as plsc

U = 100000
Q = 1024
C = 16
KNN = 50
NBINS = 403
BSTRIDE = 408
UB = 2000
NBLK = U // UB
CH = 2000
NCH = U // CH
SLOT = 64
NW = 32
L = 16


def _tc_body(uid_ref, qt_ref, r_ref, keys_ref, aux_ref, meanq_ref, zeroq_ref):
    i = pl.program_id(0)
    qt = qt_ref[...]
    mqt = (qt != 0.0).astype(jnp.float32)
    rmat = jnp.concatenate([qt * qt, -2.0 * qt, mqt], axis=0)
    r = r_ref[...]
    mr = (r != 0.0).astype(jnp.float32)
    lmat = jnp.concatenate([mr, r, r * r], axis=1)
    sq = lax.dot_general(lmat, rmat, (((1,), (0,)), ((), ())),
                         preferred_element_type=jnp.float32)
    mc = lax.dot_general(mr, mqt, (((1,), (0,)), ((), ())),
                         preferred_element_type=jnp.float32)
    key = (jnp.maximum(sq, 0.0) + 0.5).astype(jnp.int32)
    key = jnp.where(mc > 0.5, key, 401)
    rowid = i * UB + lax.broadcasted_iota(jnp.int32, (UB, Q), 0)
    key = jnp.where(rowid == uid_ref[0], 402, key)
    keys_ref[...] = key[:, :512] + key[:, 512:] * 65536

    @pl.when(i == 0)
    def _init():
        sumq = jnp.sum(qt, axis=0)
        meanq_ref[...] = (sumq * (1.0 / 16.0)).reshape(8, 128)
        zeroq_ref[...] = (sumq == 0.0).astype(jnp.float32).reshape(8, 128)
        aux_ref[...] = jnp.zeros((8, 128), jnp.float32)

    cnt = jnp.stack([jnp.sum((r == float(v)).astype(jnp.float32), axis=0)
                     for v in range(6)], axis=0)
    aux_ref[...] += jnp.pad(cnt, ((0, 2), (0, 112)))

    @pl.when(i == NBLK - 1)
    def _finish():
        a = aux_ref[...]
        bestc = a[0:1, :]
        bestv = jnp.zeros((1, 128), jnp.float32)
        for v in range(1, 6):
            cv = a[v:v + 1, :]
            take = cv > bestc
            bestv = jnp.where(take, float(v), bestv)
            bestc = jnp.maximum(bestc, cv)
        aux_ref[6:7, :] = bestv


def _tc_keys(qt, r, uid, interpret=False):
    return pl.pallas_call(
        _tc_body,
        grid=(NBLK,),
        in_specs=[
            pl.BlockSpec(memory_space=pltpu.SMEM),
            pl.BlockSpec((16, Q), lambda i: (0, 0)),
            pl.BlockSpec((UB, 16), lambda i: (i, 0)),
        ],
        out_specs=[
            pl.BlockSpec((UB, 512), lambda i: (i, 0)),
            pl.BlockSpec((8, 128), lambda i: (0, 0)),
            pl.BlockSpec((8, 128), lambda i: (0, 0)),
            pl.BlockSpec((8, 128), lambda i: (0, 0)),
        ],
        out_shape=[
            jax.ShapeDtypeStruct((U, 512), jnp.int32),
            jax.ShapeDtypeStruct((8, 128), jnp.float32),
            jax.ShapeDtypeStruct((8, 128), jnp.float32),
            jax.ShapeDtypeStruct((8, 128), jnp.float32),
        ],
        interpret=interpret,
    )(uid, qt, r)


def _sc_stage(keys, users, wlut, meanq, zeroq, mostpop):
    mesh = plsc.VectorSubcoreMesh(core_axis_name="c", subcore_axis_name="s")

    @functools.partial(
        pl.kernel,
        mesh=mesh,
        compiler_params=pltpu.CompilerParams(needs_layout_passes=False,
                                             use_tc_tiling_on_sc=False),
        out_type=jax.ShapeDtypeStruct((Q, C), jnp.float32),
        scratch_types=[
            pltpu.VMEM((CH, L), jnp.int32),
            pltpu.VMEM((CH, L), jnp.int32),
            pltpu.VMEM((NW * BSTRIDE,), jnp.int32),
            pltpu.VMEM((2 * SLOT * L,), jnp.int32),
            pltpu.VMEM((2 * SLOT * L,), jnp.int32),
            pltpu.VMEM((2 * SLOT * L,), jnp.int32),
            pltpu.VMEM((KNN, C), jnp.float32),
            pltpu.VMEM((BSTRIDE,), jnp.float32),
            pltpu.VMEM((Q,), jnp.float32),
            pltpu.VMEM((Q,), jnp.float32),
            pltpu.VMEM((L,), jnp.float32),
            pltpu.VMEM((L, C), jnp.float32),
            pltpu.VMEM((L, C), jnp.float32),
            pltpu.SemaphoreType.DMA,
            pltpu.SemaphoreType.DMA,
            pltpu.SemaphoreType.DMA,
        ],
    )
    def body(keys_hbm, users_hbm, wlut_hbm, meanq_hbm, zeroq_hbm, mp_hbm,
             out_hbm, buf0, buf1, hist, selbuf, idxall, keyall, rows, wl,
             mqv, zqv, mpv, preda, predb, sem0, sem1, semg):
        def sload(ref, idx):
            v = plsc.load_gather(ref, [jnp.zeros((16,), jnp.int32) + idx])
            return v[0]

        wid = lax.axis_index("s") * 2 + lax.axis_index("c")
        col0 = wid * L
        lane = lax.iota(jnp.int32, 16)
        ones = jnp.ones((16,), jnp.int32)
        offa = lane * BSTRIDE
        offb = offa + L * BSTRIDE

        pltpu.sync_copy(wlut_hbm, wl)
        pltpu.sync_copy(meanq_hbm, mqv)
        pltpu.sync_copy(zeroq_hbm, zqv)
        pltpu.sync_copy(mp_hbm, mpv)

        def zinit(j, _):
            hist[pl.ds(j * 16, 16)] = jnp.zeros((16,), jnp.int32)
            return 0
        lax.fori_loop(0, NW * BSTRIDE // 16, zinit, 0)

        bufs = (buf0, buf1)
        sems = (sem0, sem1)

        def dma(c, b):
            return pltpu.make_async_copy(
                keys_hbm.at[wid, pl.ds(c * CH, CH), :],
                bufs[b], sems[b])

        dma(0, 0).start()
        dma(1, 1).start()

        def hist_user(buf):
            def step(j, _):
                k32 = buf[j]
                ka = k32 & 0xFFFF
                kb = lax.shift_right_logical(k32, 16)
                plsc.addupdate_scatter(hist, [offa + ka], ones)
                plsc.addupdate_scatter(hist, [offb + kb], ones)
                return 0
            lax.fori_loop(0, CH, step, 0)

        def hist_chunk(c2, _):
            for b in range(2):
                cc = c2 * 2 + b
                dma(cc, b).wait()
                hist_user(bufs[b])
                nxt = cc + 2

                @pl.when(nxt < NCH)
                def _():
                    dma(nxt, b).start()
            return 0
        lax.fori_loop(0, NCH // 2, hist_chunk, 0)

        def scan_half(off):
            def sbody(b, st):
                cum, tt, cl = st
                h = plsc.load_gather(hist, [off + b])
                ncum = cum + h
                newly = (tt < 0) & (ncum >= KNN)
                tt = jnp.where(newly, b, tt)
                cl = jnp.where(newly, cum, cl)
                return (ncum, tt, cl)
            z = jnp.zeros((16,), jnp.int32)
            _, tt, cl = lax.fori_loop(0, NBINS, sbody,
                                      (z, z - 1, z))
            return tt, (KNN - cl)
        ta, ma = scan_half(offa)
        tb, mb = scan_half(offb)

        dma(0, 0).start()
        dma(1, 1).start()

        def sel_users(buf, u0, st):
            def step(j, st):
                pa, pb, ea, eb = st
                k32 = buf[j]
                u = u0 + j
                ka = k32 & 0xFFFF
                kb = lax.shift_right_logical(k32, 16)
                eqa = ka == ta
                sela = (ka < ta) | (eqa & (ea < ma))
                eqb = kb == tb
                selb = (kb < tb) | (eqb & (eb < mb))
                pka = lax.shift_left(ka, 17) + u
                pkb = lax.shift_left(kb, 17) + u
                plsc.store_scatter(selbuf, [pa], pka, mask=sela)
                plsc.store_scatter(selbuf, [pb], pkb, mask=selb)
                pa = pa + sela.astype(jnp.int32)
                pb = pb + selb.astype(jnp.int32)
                ea = ea + eqa.astype(jnp.int32)
                eb = eb + eqb.astype(jnp.int32)
                return (pa, pb, ea, eb)
            return lax.fori_loop(0, CH, step, st)

        def sel_chunk(c2, st):
            for b in range(2):
                cc = c2 * 2 + b
                dma(cc, b).wait()
                st = sel_users(bufs[b], cc * CH, st)
                nxt = cc + 2

                @pl.when(nxt < NCH)
                def _():
                    dma(nxt, b).start()
            return st
        z = jnp.zeros((16,), jnp.int32)
        lax.fori_loop(0, NCH // 2, sel_chunk,
                      (lane * SLOT, lane * SLOT + L * SLOT, z, z))

        def unp(t, _):
            v = selbuf[pl.ds(t * 16, 16)]
            idxall[pl.ds(t * 16, 16)] = v & 0x1FFFF
            keyall[pl.ds(t * 16, 16)] = lax.shift_right_logical(v, 17)
            return 0
        lax.fori_loop(0, 2 * SLOT * L // 16, unp, 0)

        def aggregate(l, _):
            pltpu.async_copy(
                users_hbm.at[idxall.at[pl.ds(l * SLOT, KNN)]],
                rows, semg).wait()

            def nstep(n, st):
                acc, sumw = st
                row = rows[n]
                w = sload(wl, sload(keyall, l * SLOT + n))
                nm = jnp.sum(row) * (1.0 / 16.0)
                return (acc + w * (row - nm), sumw + w)
            acc, sumw = lax.fori_loop(
                0, KNN, nstep, (jnp.zeros((16,), jnp.float32),
                                jnp.float32(0.0)))

            qi = col0 + jnp.where(l < L, l, 512 - L + l)
            pred = sload(mqv, qi) + acc / sumw
            pred = jnp.clip(pred, 0.0, 5.0)
            pred = jnp.where(sload(zqv, qi) > 0.5, mpv[...], pred)
            ll = jnp.where(l < L, l, l - L)

            @pl.when(l < L)
            def _():
                plsc.store_scatter(preda, [jnp.full((16,), 0, jnp.int32) + ll,
                                           lane], pred)

            @pl.when(l >= L)
            def _():
                plsc.store_scatter(predb, [jnp.full((16,), 0, jnp.int32) + ll,
                                           lane], pred)
            return 0
        lax.fori_loop(0, 2 * L, aggregate, 0)

        pltpu.sync_copy(preda, out_hbm.at[pl.ds(col0, L), :])
        pltpu.sync_copy(predb, out_hbm.at[pl.ds(512 + col0, L), :])

    return body(keys, users, wlut, meanq, zeroq, mostpop)


def kernel(input, users_matrix, user_id):
    qt = jnp.transpose(input)
    uid = jnp.asarray(user_id, jnp.int32).reshape(1)
    keys, aux, meanq, zeroq = _tc_keys(qt, users_matrix, uid)
    wlut = jnp.concatenate([
        1.0 / (1.0 + jnp.sqrt(jnp.arange(401, dtype=jnp.float32))),
        jnp.array([-1000.0, -3000.0], jnp.float32),
        jnp.zeros((BSTRIDE - NBINS,), jnp.float32)])
    keysw = jnp.transpose(keys.reshape(U, NW, L), (1, 0, 2))
    pred = _sc_stage(keysw, users_matrix, wlut,
                     meanq.reshape(Q), zeroq.reshape(Q), aux[6, :16])
    return pred
